# baseline (speedup 1.0000x reference)
import functools

import jax
import jax.numpy as jnp
from jax import lax
from jax.experimental import pallas as pl
from jax.experimental.pallas import tpu as pltpu
from jax.experimental.pallas import tpu_sc as plsc

_NUM_WORKERS = 32
_LANES = 16
_BLK = 128


def _pick_chunk(total, cap):
    c = cap
    while c >= _LANES:
        if total % c == 0 and c % _LANES == 0:
            return c
        c -= _LANES
    raise ValueError(f"no chunk for {total}")


def _pick_cb(blocks_lo, cap):
    for cb in range(cap, 0, -1):
        if blocks_lo % cb == 0:
            return cb
    return 1


@functools.lru_cache(maxsize=None)
def _build(n_types_pad, n_atoms, n_pairs, interpret=False):
    assert n_pairs % _BLK == 0
    nb = n_pairs // _BLK
    blocks_lo = nb // _NUM_WORKERS
    n_tail = nb - blocks_lo * _NUM_WORKERS
    CB = _pick_cb(blocks_lo, 22)
    n_chunks = blocks_lo // CB
    CP = CB * _BLK
    AC = _pick_chunk(n_atoms, 4000)
    n_achunks = n_atoms // AC
    assert AC <= 2 * CP

    mesh = plsc.VectorSubcoreMesh(core_axis_name="c", subcore_axis_name="s")

    @functools.partial(
        pl.kernel,
        out_type=jax.ShapeDtypeStruct((n_pairs,), jnp.float32),
        mesh=mesh,
        scratch_types=[
            pltpu.VMEM((n_types_pad,), jnp.float32),
            pltpu.VMEM((n_atoms,), jnp.float32),
            pltpu.VMEM((2 * CP,), jnp.int32),
            pltpu.VMEM((CP,), jnp.float32),
            pltpu.VMEM((CP,), jnp.float32),
        ],
        compiler_params=pltpu.CompilerParams(
            needs_layout_passes=False, use_tc_tiling_on_sc=False
        ),
        interpret=interpret,
    )
    def vdw_kernel(vdw_hbm, anum_hbm, idx_hbm, dist_hbm, out_hbm,
                   vdw_v, rad_v, idx_v, dist_v, outc_v):
        wid = lax.axis_index("s") * 2 + lax.axis_index("c")
        pltpu.sync_copy(vdw_hbm, vdw_v)

        def atom_chunk(c, _):
            pltpu.sync_copy(anum_hbm.at[pl.ds(c * AC, AC)],
                            idx_v.at[pl.ds(0, AC)])
            def grp(g, _):
                nums = idx_v[pl.ds(g * _LANES, _LANES)]
                rad = plsc.load_gather(vdw_v, [nums])
                rad_v[pl.ds(c * AC + g * _LANES, _LANES)] = rad
                return 0
            return lax.fori_loop(0, AC // _LANES, grp, 0, unroll=False)
        lax.fori_loop(0, n_achunks, atom_chunk, 0, unroll=False)

        b0 = wid * blocks_lo + lax.min(wid, n_tail)

        def do_blocks(idx_word_off, pair_off, nblocks):
            def blk(bb, _):
                ibase = bb * (2 * _BLK)
                pbase = bb * _BLK
                def grp(r, _):
                    ii = idx_v[pl.ds(ibase + r * _LANES, _LANES)]
                    jj = idx_v[pl.ds(ibase + _BLK + r * _LANES, _LANES)]
                    ri = plsc.load_gather(rad_v, [ii])
                    rj = plsc.load_gather(rad_v, [jj])
                    d = dist_v[pl.ds(pbase + r * _LANES, _LANES)]
                    outc_v[pl.ds(pbase + r * _LANES, _LANES)] = (
                        (ri + rj) * 0.5 / d)
                    return 0
                return lax.fori_loop(0, _BLK // _LANES, grp, 0, unroll=False)
            lax.fori_loop(0, nblocks, blk, 0, unroll=False)

        def pair_chunk(c, _):
            boff = b0 + c * CB
            pltpu.sync_copy(idx_hbm.at[pl.ds(boff * 2 * _BLK, 2 * CP)], idx_v)
            pltpu.sync_copy(dist_hbm.at[pl.ds(boff * _BLK, CP)], dist_v)
            do_blocks(0, 0, CB)
            pltpu.sync_copy(outc_v, out_hbm.at[pl.ds(boff * _BLK, CP)])
            return 0
        lax.fori_loop(0, n_chunks, pair_chunk, 0, unroll=False)

        @pl.when(wid < n_tail)
        def _tail():
            boff = b0 + blocks_lo
            pltpu.sync_copy(idx_hbm.at[pl.ds(boff * 2 * _BLK, 2 * _BLK)],
                            idx_v.at[pl.ds(0, 2 * _BLK)])
            pltpu.sync_copy(dist_hbm.at[pl.ds(boff * _BLK, _BLK)],
                            dist_v.at[pl.ds(0, _BLK)])
            do_blocks(0, 0, 1)
            pltpu.sync_copy(outc_v.at[pl.ds(0, _BLK)],
                            out_hbm.at[pl.ds(boff * _BLK, _BLK)])

    return vdw_kernel


def kernel(atom_vdw, atoms_long, batch_atom_ij_idx, batch_dist_ij):
    n_types = atom_vdw.shape[0]
    n_pairs = batch_dist_ij.shape[0]
    n_types_pad = max(128, -(-n_types // 8) * 8)
    vdw_pad = jnp.zeros((n_types_pad,), jnp.float32).at[:n_types].set(atom_vdw)
    anum = atoms_long[:, 1]
    idx_flat = (
        batch_atom_ij_idx.reshape(n_pairs // _BLK, _BLK, 2)
        .transpose(0, 2, 1)
        .reshape(-1)
    )
    fn = _build(n_types_pad, atoms_long.shape[0], n_pairs)
    return fn(vdw_pad, anum, idx_flat, batch_dist_ij)


# speedup vs baseline: 1.7411x; 1.7411x over previous
import functools

import jax
import jax.numpy as jnp
from jax import lax
from jax.experimental import pallas as pl
from jax.experimental.pallas import tpu as pltpu
from jax.experimental.pallas import tpu_sc as plsc

_NUM_WORKERS = 32
_LANES = 16
_BLK = 128


def _pick_chunk(total, cap):
    c = cap
    while c >= _LANES:
        if total % c == 0 and c % _LANES == 0:
            return c
        c -= _LANES
    raise ValueError(f"no chunk for {total}")


def _pick_cb(blocks_lo, cap):
    for cb in range(cap, 0, -1):
        if blocks_lo % cb == 0:
            return cb
    return 1


@functools.lru_cache(maxsize=None)
def _build(n_types_pad, n_atoms, n_pairs, interpret=False):
    assert n_pairs % _BLK == 0
    nb = n_pairs // _BLK
    blocks_lo = nb // _NUM_WORKERS
    n_tail = nb - blocks_lo * _NUM_WORKERS
    CB = _pick_cb(blocks_lo, 22)
    n_chunks = blocks_lo // CB
    CP = CB * _BLK
    AC = _pick_chunk(n_atoms, 4000)
    n_achunks = n_atoms // AC
    assert AC <= 2 * CP

    mesh = plsc.VectorSubcoreMesh(core_axis_name="c", subcore_axis_name="s")

    @functools.partial(
        pl.kernel,
        out_type=jax.ShapeDtypeStruct((n_pairs,), jnp.float32),
        mesh=mesh,
        scratch_types=[
            pltpu.VMEM((n_types_pad,), jnp.float32),
            pltpu.VMEM((n_atoms,), jnp.float32),
            pltpu.VMEM((2 * CP,), jnp.int32),
            pltpu.VMEM((CP,), jnp.float32),
            pltpu.VMEM((CP,), jnp.float32),
        ],
        compiler_params=pltpu.CompilerParams(
            needs_layout_passes=False, use_tc_tiling_on_sc=False
        ),
        interpret=interpret,
    )
    def vdw_kernel(vdw_hbm, anum_hbm, idx_hbm, dist_hbm, out_hbm,
                   vdw_v, rad_v, idx_v, dist_v, outc_v):
        wid = lax.axis_index("s") * 2 + lax.axis_index("c")
        pltpu.sync_copy(vdw_hbm, vdw_v)

        def atom_chunk(c, _):
            pltpu.sync_copy(anum_hbm.at[pl.ds(c * AC, AC)],
                            idx_v.at[pl.ds(0, AC)])
            @plsc.parallel_loop(0, AC // _LANES, unroll=4)
            def grp(g):
                nums = idx_v[pl.ds(g * _LANES, _LANES)]
                rad = plsc.load_gather(vdw_v, [nums])
                rad_v[pl.ds(c * AC + g * _LANES, _LANES)] = rad
            return 0
        lax.fori_loop(0, n_achunks, atom_chunk, 0, unroll=False)

        b0 = wid * blocks_lo + lax.min(wid, n_tail)

        def do_blocks(idx_word_off, pair_off, nblocks):
            @plsc.parallel_loop(0, nblocks, unroll=2)
            def blk(bb):
                ibase = bb * (2 * _BLK)
                pbase = bb * _BLK
                for r in range(_BLK // _LANES):
                    ii = idx_v[pl.ds(ibase + r * _LANES, _LANES)]
                    jj = idx_v[pl.ds(ibase + _BLK + r * _LANES, _LANES)]
                    ri = plsc.load_gather(rad_v, [ii])
                    rj = plsc.load_gather(rad_v, [jj])
                    d = dist_v[pl.ds(pbase + r * _LANES, _LANES)]
                    outc_v[pl.ds(pbase + r * _LANES, _LANES)] = (
                        (ri + rj) * 0.5 / d)

        def pair_chunk(c, _):
            boff = b0 + c * CB
            pltpu.sync_copy(idx_hbm.at[pl.ds(boff * 2 * _BLK, 2 * CP)], idx_v)
            pltpu.sync_copy(dist_hbm.at[pl.ds(boff * _BLK, CP)], dist_v)
            do_blocks(0, 0, CB)
            pltpu.sync_copy(outc_v, out_hbm.at[pl.ds(boff * _BLK, CP)])
            return 0
        lax.fori_loop(0, n_chunks, pair_chunk, 0, unroll=False)

        @pl.when(wid < n_tail)
        def _tail():
            boff = b0 + blocks_lo
            pltpu.sync_copy(idx_hbm.at[pl.ds(boff * 2 * _BLK, 2 * _BLK)],
                            idx_v.at[pl.ds(0, 2 * _BLK)])
            pltpu.sync_copy(dist_hbm.at[pl.ds(boff * _BLK, _BLK)],
                            dist_v.at[pl.ds(0, _BLK)])
            do_blocks(0, 0, 1)
            pltpu.sync_copy(outc_v.at[pl.ds(0, _BLK)],
                            out_hbm.at[pl.ds(boff * _BLK, _BLK)])

    return vdw_kernel


def kernel(atom_vdw, atoms_long, batch_atom_ij_idx, batch_dist_ij):
    n_types = atom_vdw.shape[0]
    n_pairs = batch_dist_ij.shape[0]
    n_types_pad = max(128, -(-n_types // 8) * 8)
    vdw_pad = jnp.zeros((n_types_pad,), jnp.float32).at[:n_types].set(atom_vdw)
    anum = atoms_long[:, 1]
    idx_flat = (
        batch_atom_ij_idx.reshape(n_pairs // _BLK, _BLK, 2)
        .transpose(0, 2, 1)
        .reshape(-1)
    )
    fn = _build(n_types_pad, atoms_long.shape[0], n_pairs)
    return fn(vdw_pad, anum, idx_flat, batch_dist_ij)


# speedup vs baseline: 1.9621x; 1.1269x over previous
import functools

import jax
import jax.numpy as jnp
from jax import lax
from jax.experimental import pallas as pl
from jax.experimental.pallas import tpu as pltpu
from jax.experimental.pallas import tpu_sc as plsc

_NUM_WORKERS = 32
_LANES = 16
_BLK = 128


def _pick_chunk(total, cap):
    c = cap
    while c >= _LANES:
        if total % c == 0 and c % _LANES == 0:
            return c
        c -= _LANES
    raise ValueError(f"no chunk for {total}")


def _pick_cb(blocks_lo, cap):
    for cb in range(cap, 0, -1):
        if blocks_lo % cb == 0:
            return cb
    return 1


@functools.lru_cache(maxsize=None)
def _build(n_types_pad, n_atoms, n_pairs, interpret=False):
    assert n_pairs % _BLK == 0
    nb = n_pairs // _BLK
    blocks_lo = nb // _NUM_WORKERS
    n_tail = nb - blocks_lo * _NUM_WORKERS
    CB = _pick_cb(blocks_lo, 22)
    n_chunks = blocks_lo // CB
    CP = CB * _BLK
    assert n_atoms % _LANES == 0
    n_grp = n_atoms // _LANES
    GS = -(-n_grp // 16)
    AC = GS * _LANES
    IDXSZ = max(2 * CP, AC)

    mesh = plsc.VectorSubcoreMesh(core_axis_name="c", subcore_axis_name="s")

    @functools.partial(
        pl.kernel,
        out_type=jax.ShapeDtypeStruct((n_pairs,), jnp.float32),
        mesh=mesh,
        scratch_types=[
            pltpu.VMEM((n_types_pad,), jnp.float32),
            pltpu.VMEM((n_atoms,), jnp.float32),
            pltpu.VMEM((IDXSZ,), jnp.int32),
            pltpu.VMEM((CP,), jnp.float32),
            pltpu.VMEM((CP,), jnp.float32),
            pltpu.VMEM_SHARED((n_atoms,), jnp.float32),
        ],
        compiler_params=pltpu.CompilerParams(
            needs_layout_passes=False, use_tc_tiling_on_sc=False
        ),
        interpret=interpret,
    )
    def vdw_kernel(vdw_hbm, anum_hbm, idx_hbm, dist_hbm, out_hbm,
                   vdw_v, rad_v, idx_v, dist_v, outc_v, rad_sh):
        sid = lax.axis_index("s")
        wid = sid * 2 + lax.axis_index("c")
        pltpu.sync_copy(vdw_hbm, vdw_v)

        a0 = lax.min(sid * GS, n_grp - GS) * _LANES
        pltpu.sync_copy(anum_hbm.at[pl.ds(a0, AC)], idx_v.at[pl.ds(0, AC)])

        @plsc.parallel_loop(0, GS, unroll=4)
        def grp(g):
            nums = idx_v[pl.ds(g * _LANES, _LANES)]
            rad = plsc.load_gather(vdw_v, [nums])
            rad_v[pl.ds(a0 + g * _LANES, _LANES)] = rad

        pltpu.sync_copy(rad_v.at[pl.ds(a0, AC)], rad_sh.at[pl.ds(a0, AC)])
        plsc.subcore_barrier()
        pltpu.sync_copy(rad_sh, rad_v)

        b0 = wid * blocks_lo + lax.min(wid, n_tail)

        def do_blocks(idx_word_off, pair_off, nblocks):
            @plsc.parallel_loop(0, nblocks, unroll=2)
            def blk(bb):
                ibase = bb * (2 * _BLK)
                pbase = bb * _BLK
                for r in range(_BLK // _LANES):
                    ii = idx_v[pl.ds(ibase + r * _LANES, _LANES)]
                    jj = idx_v[pl.ds(ibase + _BLK + r * _LANES, _LANES)]
                    ri = plsc.load_gather(rad_v, [ii])
                    rj = plsc.load_gather(rad_v, [jj])
                    d = dist_v[pl.ds(pbase + r * _LANES, _LANES)]
                    outc_v[pl.ds(pbase + r * _LANES, _LANES)] = (
                        (ri + rj) * 0.5 / d)

        def pair_chunk(c, _):
            boff = b0 + c * CB
            pltpu.sync_copy(idx_hbm.at[pl.ds(boff * 2 * _BLK, 2 * CP)],
                            idx_v.at[pl.ds(0, 2 * CP)])
            pltpu.sync_copy(dist_hbm.at[pl.ds(boff * _BLK, CP)], dist_v)
            do_blocks(0, 0, CB)
            pltpu.sync_copy(outc_v, out_hbm.at[pl.ds(boff * _BLK, CP)])
            return 0
        lax.fori_loop(0, n_chunks, pair_chunk, 0, unroll=False)

        @pl.when(wid < n_tail)
        def _tail():
            boff = b0 + blocks_lo
            pltpu.sync_copy(idx_hbm.at[pl.ds(boff * 2 * _BLK, 2 * _BLK)],
                            idx_v.at[pl.ds(0, 2 * _BLK)])
            pltpu.sync_copy(dist_hbm.at[pl.ds(boff * _BLK, _BLK)],
                            dist_v.at[pl.ds(0, _BLK)])
            do_blocks(0, 0, 1)
            pltpu.sync_copy(outc_v.at[pl.ds(0, _BLK)],
                            out_hbm.at[pl.ds(boff * _BLK, _BLK)])

    return vdw_kernel


def kernel(atom_vdw, atoms_long, batch_atom_ij_idx, batch_dist_ij):
    n_types = atom_vdw.shape[0]
    n_pairs = batch_dist_ij.shape[0]
    n_types_pad = max(128, -(-n_types // 8) * 8)
    vdw_pad = jnp.zeros((n_types_pad,), jnp.float32).at[:n_types].set(atom_vdw)
    anum = atoms_long[:, 1]
    idx_flat = (
        batch_atom_ij_idx.reshape(n_pairs // _BLK, _BLK, 2)
        .transpose(0, 2, 1)
        .reshape(-1)
    )
    fn = _build(n_types_pad, atoms_long.shape[0], n_pairs)
    return fn(vdw_pad, anum, idx_flat, batch_dist_ij)


# speedup vs baseline: 4.1348x; 2.1074x over previous
import functools

import jax
import jax.numpy as jnp
from jax import lax
from jax.experimental import pallas as pl
from jax.experimental.pallas import tpu as pltpu
from jax.experimental.pallas import tpu_sc as plsc

_NUM_WORKERS = 32
_LANES = 16
_BLK = 128


def _pick_chunk(total, cap):
    c = cap
    while c >= _LANES:
        if total % c == 0 and c % _LANES == 0:
            return c
        c -= _LANES
    raise ValueError(f"no chunk for {total}")


def _pick_cb(blocks_lo, cap):
    for cb in range(cap, 0, -1):
        if blocks_lo % cb == 0:
            return cb
    return 1


@functools.lru_cache(maxsize=None)
def _build(n_types_pad, n_atoms, n_pairs, interpret=False):
    assert n_pairs % _BLK == 0
    nb = n_pairs // _BLK
    blocks_lo = nb // _NUM_WORKERS
    n_tail = nb - blocks_lo * _NUM_WORKERS
    CB = _pick_cb(blocks_lo, 22)
    n_chunks = blocks_lo // CB
    CP = CB * _BLK
    assert n_atoms % _LANES == 0
    n_grp = n_atoms // _LANES
    GS = -(-n_grp // 16)
    AC = GS * _LANES
    IDXSZ = max(2 * CP, AC)

    mesh = plsc.VectorSubcoreMesh(core_axis_name="c", subcore_axis_name="s")

    @functools.partial(
        pl.kernel,
        out_type=jax.ShapeDtypeStruct((n_pairs,), jnp.float32),
        mesh=mesh,
        scratch_types=[
            pltpu.VMEM((n_types_pad,), jnp.float32),
            pltpu.VMEM((n_atoms,), jnp.float32),
            pltpu.VMEM((IDXSZ,), jnp.int32),
            pltpu.VMEM((2 * CP,), jnp.int32),
            pltpu.VMEM((CP,), jnp.float32),
            pltpu.VMEM((CP,), jnp.float32),
            pltpu.VMEM((CP,), jnp.float32),
            pltpu.VMEM((CP,), jnp.float32),
            pltpu.VMEM_SHARED((n_atoms,), jnp.float32),
            pltpu.SemaphoreType.DMA,
            pltpu.SemaphoreType.DMA,
            pltpu.SemaphoreType.DMA,
            pltpu.SemaphoreType.DMA,
        ],
        compiler_params=pltpu.CompilerParams(
            needs_layout_passes=False, use_tc_tiling_on_sc=False
        ),
        interpret=interpret,
    )
    def vdw_kernel(vdw_hbm, anum_hbm, idx_hbm, dist_hbm, out_hbm,
                   vdw_v, rad_v, idx_v, idx_v1, dist_v, dist_v1,
                   outc_v, outc_v1, rad_sh,
                   sem_i0, sem_i1, sem_o0, sem_o1):
        sid = lax.axis_index("s")
        wid = sid * 2 + lax.axis_index("c")
        pltpu.sync_copy(vdw_hbm, vdw_v)

        a0 = lax.min(sid * GS, n_grp - GS) * _LANES
        pltpu.sync_copy(anum_hbm.at[pl.ds(a0, AC)], idx_v.at[pl.ds(0, AC)])

        @plsc.parallel_loop(0, GS, unroll=4)
        def grp(g):
            nums = idx_v[pl.ds(g * _LANES, _LANES)]
            rad = plsc.load_gather(vdw_v, [nums])
            rad_v[pl.ds(a0 + g * _LANES, _LANES)] = rad

        pltpu.sync_copy(rad_v.at[pl.ds(a0, AC)], rad_sh.at[pl.ds(a0, AC)])
        plsc.subcore_barrier()
        pltpu.sync_copy(rad_sh, rad_v)

        b0 = wid * blocks_lo + lax.min(wid, n_tail)
        bufs = ((idx_v, dist_v, outc_v, sem_i0, sem_o0),
                (idx_v1, dist_v1, outc_v1, sem_i1, sem_o1))

        def in_copies(c, buf):
            iv, dv, _, si, _ = buf
            boff = b0 + c * CB
            return (
                pltpu.make_async_copy(
                    idx_hbm.at[pl.ds(boff * 2 * _BLK, 2 * CP)],
                    iv.at[pl.ds(0, 2 * CP)], si),
                pltpu.make_async_copy(
                    dist_hbm.at[pl.ds(boff * _BLK, CP)], dv, si),
            )

        def out_copy(c, buf):
            _, _, ov, _, so = buf
            boff = b0 + c * CB
            return pltpu.make_async_copy(
                ov, out_hbm.at[pl.ds(boff * _BLK, CP)], so)

        def start_in(c, buf):
            for cp in in_copies(c, buf):
                cp.start()

        def wait_in(c, buf):
            for cp in in_copies(c, buf):
                cp.wait()

        def do_blocks(buf, nblocks):
            iv, dv, ov, _, _ = buf
            @plsc.parallel_loop(0, nblocks, unroll=2)
            def blk(bb):
                ibase = bb * (2 * _BLK)
                pbase = bb * _BLK
                for r in range(_BLK // _LANES):
                    ii = iv[pl.ds(ibase + r * _LANES, _LANES)]
                    jj = iv[pl.ds(ibase + _BLK + r * _LANES, _LANES)]
                    ri = plsc.load_gather(rad_v, [ii])
                    rj = plsc.load_gather(rad_v, [jj])
                    d = dv[pl.ds(pbase + r * _LANES, _LANES)]
                    ov[pl.ds(pbase + r * _LANES, _LANES)] = (
                        (ri + rj) * 0.5 / d)

        def half(i, c, buf):
            wait_in(c, buf)
            @pl.when(i > 0)
            def _():
                out_copy(c, buf).wait()
            do_blocks(buf, CB)
            out_copy(c, buf).start()
            @pl.when(c + 2 < n_chunks)
            def _():
                start_in(c + 2, buf)

        start_in(0, bufs[0])
        if n_chunks > 1:
            start_in(1, bufs[1])

        def pair_two(i, _):
            half(i, 2 * i, bufs[0])
            half(i, 2 * i + 1, bufs[1])
            return 0
        lax.fori_loop(0, n_chunks // 2, pair_two, 0, unroll=False)

        if n_chunks % 2:
            c_last = n_chunks - 1
            wait_in(c_last, bufs[0])
            if n_chunks > 1:
                out_copy(c_last, bufs[0]).wait()
            do_blocks(bufs[0], CB)
            out_copy(c_last, bufs[0]).start()
        out_copy(n_chunks - 1, bufs[0]).wait()
        if n_chunks > 1:
            out_copy(n_chunks - 2, bufs[1]).wait()

        @pl.when(wid < n_tail)
        def _tail():
            boff = b0 + blocks_lo
            pltpu.sync_copy(idx_hbm.at[pl.ds(boff * 2 * _BLK, 2 * _BLK)],
                            idx_v.at[pl.ds(0, 2 * _BLK)])
            pltpu.sync_copy(dist_hbm.at[pl.ds(boff * _BLK, _BLK)],
                            dist_v.at[pl.ds(0, _BLK)])
            do_blocks(bufs[0], 1)
            pltpu.sync_copy(outc_v.at[pl.ds(0, _BLK)],
                            out_hbm.at[pl.ds(boff * _BLK, _BLK)])

    return vdw_kernel


def kernel(atom_vdw, atoms_long, batch_atom_ij_idx, batch_dist_ij):
    n_types = atom_vdw.shape[0]
    n_pairs = batch_dist_ij.shape[0]
    n_types_pad = max(128, -(-n_types // 8) * 8)
    vdw_pad = jnp.zeros((n_types_pad,), jnp.float32).at[:n_types].set(atom_vdw)
    anum = atoms_long[:, 1]
    idx_flat = (
        batch_atom_ij_idx.reshape(n_pairs // _BLK, _BLK, 2)
        .transpose(0, 2, 1)
        .reshape(-1)
    )
    fn = _build(n_types_pad, atoms_long.shape[0], n_pairs)
    return fn(vdw_pad, anum, idx_flat, batch_dist_ij)
